# baseline (speedup 1.0000x reference)
import functools

import numpy as np
import jax
import jax.numpy as jnp
from jax import lax
from jax.experimental import pallas as pl
from jax.experimental.pallas import tpu as pltpu
from jax.experimental.pallas import tpu_sc as plsc

FIELD = 26
EMB = 16

_NC = 2
_NS = 16
_NW = _NC * _NS

_POOL = np.tile(np.eye(EMB, dtype=np.float32), (FIELD, 1))


def _sc_transpose(tcol1d, fcol1d, v):
    vc = 2000
    n_chunks_total = v // vc
    base_chunks = n_chunks_total // _NW
    extra = n_chunks_total - base_chunks * _NW
    mesh = plsc.VectorSubcoreMesh(core_axis_name="c", subcore_axis_name="s")

    @functools.partial(
        pl.kernel,
        mesh=mesh,
        compiler_params=pltpu.CompilerParams(use_tc_tiling_on_sc=False,
                                             needs_layout_passes=False),
        out_type=[
            jax.ShapeDtypeStruct((v * EMB,), jnp.float32),
            jax.ShapeDtypeStruct((v,), jnp.float32),
        ],
        scratch_types=[
            pltpu.VMEM((EMB, vc), jnp.float32),
            pltpu.VMEM((vc * EMB,), jnp.float32),
            pltpu.VMEM((vc,), jnp.float32),
            pltpu.SemaphoreType.DMA,
            pltpu.SemaphoreType.DMA,
        ],
    )
    def body(tcol_hbm, fcol_hbm, tbl_out, fst_out, colbuf, outbuf, fbuf,
             sem_c, sem_f):
        wid = lax.axis_index("s") * _NC + lax.axis_index("c")
        n_c = base_chunks + jnp.where(wid < extra, 1, 0)
        lane_iota = lax.iota(jnp.int32, 16)

        def chunk_body(c, carry):
            v0 = (wid + c * _NW) * vc
            copies = [pltpu.async_copy(
                fcol_hbm.at[pl.ds(v0, vc)], fbuf, sem_f)]
            for e in range(EMB):
                copies.append(pltpu.async_copy(
                    tcol_hbm.at[pl.ds(e * v + v0, vc)], colbuf.at[e], sem_c))
            for cp in copies:
                cp.wait()

            def m_body(m, carry2):
                vals = plsc.load_gather(
                    colbuf, [lane_iota, jnp.full((16,), 0, jnp.int32) + m])
                outbuf[pl.ds(m * EMB, EMB)] = vals
                return carry2

            lax.fori_loop(0, vc, m_body, 0)
            pltpu.sync_copy(outbuf, tbl_out.at[pl.ds(v0 * EMB, vc * EMB)])
            pltpu.sync_copy(fbuf, fst_out.at[pl.ds(v0, vc)])
            return carry

        lax.fori_loop(0, n_c, chunk_body, 0)

    return body(tcol1d, fcol1d)


def _sc_gather(xf2d, sec, fst, total):
    per_w = total // _NW
    chunk = 1024
    ng = chunk // 128
    n_chunks = per_w // chunk
    mesh = plsc.VectorSubcoreMesh(core_axis_name="c", subcore_axis_name="s")

    @functools.partial(
        pl.kernel,
        mesh=mesh,
        compiler_params=pltpu.CompilerParams(use_tc_tiling_on_sc=False),
        out_type=[
            jax.ShapeDtypeStruct((total, EMB), jnp.float32),
            jax.ShapeDtypeStruct((total,), jnp.float32),
        ],
        scratch_types=[
            pltpu.VMEM((ng, 128), jnp.int32),
            pltpu.VMEM((chunk, EMB), jnp.float32),
            pltpu.VMEM((chunk,), jnp.float32),
            pltpu.SemaphoreType.DMA,
            pltpu.SemaphoreType.DMA,
        ],
    )
    def body(xf_hbm, sec_hbm, fst_hbm, demb_out, fst_out,
             idx_v, rows_v, fv_v, sem_r, sem_f):
        wid = lax.axis_index("s") * _NC + lax.axis_index("c")
        base = wid * per_w

        def chunk_body(c, carry):
            off = base + c * chunk
            row_off = pl.multiple_of(off // 128, 8)
            pltpu.sync_copy(xf_hbm.at[pl.ds(row_off, ng)], idx_v)
            copies = []
            for j in range(ng):
                copies.append(pltpu.async_copy(
                    sec_hbm.at[idx_v.at[j]],
                    rows_v.at[pl.ds(j * 128, 128)], sem_r))
                copies.append(pltpu.async_copy(
                    fst_hbm.at[idx_v.at[j]],
                    fv_v.at[pl.ds(j * 128, 128)], sem_f))
            for cp in copies:
                cp.wait()
            pltpu.sync_copy(rows_v, demb_out.at[pl.ds(off, chunk)])
            pltpu.sync_copy(fv_v, fst_out.at[pl.ds(off, chunk)])
            return carry

        lax.fori_loop(0, n_chunks, chunk_body, 0)

    return body(xf2d, sec, fst)


def _tc1_body(demb_ref, first_ref, w1t_ref, b1_ref, pool_ref, h_ref, s12_ref):
    demb = demb_ref[...]
    pool = pool_ref[...]
    sum_vec = jnp.dot(demb, pool, preferred_element_type=jnp.float32)
    sumsq = jnp.dot(demb * demb, pool, preferred_element_type=jnp.float32)
    s2 = 0.5 * jnp.sum(sum_vec * sum_vec - sumsq, axis=1)
    s1 = jnp.sum(first_ref[...], axis=1)
    h_ref[...] = (jnp.dot(demb, w1t_ref[...],
                          preferred_element_type=jnp.float32) + b1_ref[...])
    s12_ref[...] = s1 + s2


def _tc2_body(h_ref, s12_ref, g1_ref, be1_ref, w2t_ref, b2_ref, g2_ref,
              be2_ref, bias_ref, out_ref):
    h = h_ref[...]
    m1 = jnp.mean(h, axis=0)
    v1 = jnp.mean((h - m1) ** 2, axis=0)
    h = (h - m1) / jnp.sqrt(v1 + 1e-5) * g1_ref[...] + be1_ref[...]
    h = jnp.maximum(h, 0.0)

    h2 = jnp.dot(h, w2t_ref[...], preferred_element_type=jnp.float32)
    h2 = h2 + b2_ref[...]
    m2 = jnp.mean(h2, axis=0)
    v2 = jnp.mean((h2 - m2) ** 2, axis=0)
    h2 = (h2 - m2) / jnp.sqrt(v2 + 1e-5) * g2_ref[...] + be2_ref[...]
    h2 = jnp.maximum(h2, 0.0)
    sd = jnp.sum(h2, axis=1)

    z = s12_ref[...] + sd + bias_ref[...]
    out_ref[...] = 1.0 / (1.0 + jnp.exp(-z))


def kernel(x, fm_first_w, fm_second_w, w1, b1, g1, be1, w2, b2, g2, be2, bias):
    B, F = x.shape
    total = B * F
    V = fm_second_w.shape[0]
    xf2d = x.reshape(total // 128, 128)
    tbl_lin, fst_lin = _sc_transpose(fm_second_w.T.reshape(-1),
                                     fm_first_w.T.reshape(-1), V)
    demb_flat, fvals = _sc_gather(xf2d, tbl_lin.reshape(V, EMB),
                                  fst_lin, total)
    demb = demb_flat.reshape(B, F * EMB)
    first = fvals.reshape(B, F)

    cb = 2048
    nb = B // cb
    d = F * EMB
    h, s12 = pl.pallas_call(
        _tc1_body,
        grid=(nb,),
        in_specs=[
            pl.BlockSpec((cb, d), lambda i: (i, 0)),
            pl.BlockSpec((cb, F), lambda i: (i, 0)),
            pl.BlockSpec((d, 12), lambda i: (0, 0)),
            pl.BlockSpec((12,), lambda i: (0,)),
            pl.BlockSpec((d, EMB), lambda i: (0, 0)),
        ],
        out_specs=[
            pl.BlockSpec((cb, 12), lambda i: (i, 0)),
            pl.BlockSpec((cb,), lambda i: (i,)),
        ],
        out_shape=[
            jax.ShapeDtypeStruct((B, 12), jnp.float32),
            jax.ShapeDtypeStruct((B,), jnp.float32),
        ],
    )(demb, first, w1.T, b1, jnp.asarray(_POOL))

    out = pl.pallas_call(
        _tc2_body,
        out_shape=jax.ShapeDtypeStruct((B,), jnp.float32),
    )(h, s12, g1, be1, w2.T, b2, g2, be2, bias)
    return out


# speedup vs baseline: 2.6376x; 2.6376x over previous
import functools

import numpy as np
import jax
import jax.numpy as jnp
from jax import lax
from jax.experimental import pallas as pl
from jax.experimental.pallas import tpu as pltpu
from jax.experimental.pallas import tpu_sc as plsc

FIELD = 26
EMB = 16

_NC = 2
_NS = 16
_NW = _NC * _NS

_POOL = np.tile(np.eye(EMB, dtype=np.float32), (FIELD, 1))


def _sc_transpose(tcol1d, fcol1d, v):
    vc = 2000
    n_chunks_total = v // vc
    base_chunks = n_chunks_total // _NW
    extra = n_chunks_total - base_chunks * _NW
    mesh = plsc.VectorSubcoreMesh(core_axis_name="c", subcore_axis_name="s")

    @functools.partial(
        pl.kernel,
        mesh=mesh,
        compiler_params=pltpu.CompilerParams(use_tc_tiling_on_sc=False,
                                             needs_layout_passes=False),
        out_type=[
            jax.ShapeDtypeStruct((v * EMB,), jnp.float32),
            jax.ShapeDtypeStruct((v,), jnp.float32),
        ],
        scratch_types=[
            pltpu.VMEM((EMB, vc), jnp.float32),
            pltpu.VMEM((vc * EMB,), jnp.float32),
            pltpu.VMEM((vc,), jnp.float32),
            pltpu.SemaphoreType.DMA,
            pltpu.SemaphoreType.DMA,
        ],
    )
    def body(tcol_hbm, fcol_hbm, tbl_out, fst_out, colbuf, outbuf, fbuf,
             sem_c, sem_f):
        wid = lax.axis_index("s") * _NC + lax.axis_index("c")
        n_c = base_chunks + jnp.where(wid < extra, 1, 0)
        lane_iota = lax.iota(jnp.int32, 16)

        def chunk_body(c, carry):
            v0 = (wid + c * _NW) * vc
            copies = [pltpu.async_copy(
                fcol_hbm.at[pl.ds(v0, vc)], fbuf, sem_f)]
            for e in range(EMB):
                copies.append(pltpu.async_copy(
                    tcol_hbm.at[pl.ds(e * v + v0, vc)], colbuf.at[e], sem_c))
            for cp in copies:
                cp.wait()

            def m_body(m, carry2):
                vals = plsc.load_gather(
                    colbuf, [lane_iota, jnp.full((16,), 0, jnp.int32) + m])
                outbuf[pl.ds(m * EMB, EMB)] = vals
                return carry2

            lax.fori_loop(0, vc, m_body, 0)
            pltpu.sync_copy(outbuf, tbl_out.at[pl.ds(v0 * EMB, vc * EMB)])
            pltpu.sync_copy(fbuf, fst_out.at[pl.ds(v0, vc)])
            return carry

        lax.fori_loop(0, n_c, chunk_body, 0)

    return body(tcol1d, fcol1d)


def _sc_gather(xf2d, sec, fst, total):
    per_w = total // _NW
    chunk = 1024
    ng = chunk // 128
    n_chunks = per_w // chunk
    mesh = plsc.VectorSubcoreMesh(core_axis_name="c", subcore_axis_name="s")

    @functools.partial(
        pl.kernel,
        mesh=mesh,
        compiler_params=pltpu.CompilerParams(use_tc_tiling_on_sc=False),
        out_type=[
            jax.ShapeDtypeStruct((total, EMB), jnp.float32),
            jax.ShapeDtypeStruct((total,), jnp.float32),
        ],
        scratch_types=[
            pltpu.VMEM((ng, 128), jnp.int32),
            pltpu.VMEM((chunk, EMB), jnp.float32),
            pltpu.VMEM((chunk,), jnp.float32),
            pltpu.SemaphoreType.DMA,
            pltpu.SemaphoreType.DMA,
        ],
    )
    def body(xf_hbm, sec_hbm, fst_hbm, demb_out, fst_out,
             idx_v, rows_v, fv_v, sem_r, sem_f):
        wid = lax.axis_index("s") * _NC + lax.axis_index("c")
        base = wid * per_w

        def chunk_body(c, carry):
            off = base + c * chunk
            row_off = pl.multiple_of(off // 128, 8)
            pltpu.sync_copy(xf_hbm.at[pl.ds(row_off, ng)], idx_v)
            copies = []
            for j in range(ng):
                copies.append(pltpu.async_copy(
                    sec_hbm.at[idx_v.at[j]],
                    rows_v.at[pl.ds(j * 128, 128)], sem_r))
                copies.append(pltpu.async_copy(
                    fst_hbm.at[idx_v.at[j]],
                    fv_v.at[pl.ds(j * 128, 128)], sem_f))
            for cp in copies:
                cp.wait()
            pltpu.sync_copy(rows_v, demb_out.at[pl.ds(off, chunk)])
            pltpu.sync_copy(fv_v, fst_out.at[pl.ds(off, chunk)])
            return carry

        lax.fori_loop(0, n_chunks, chunk_body, 0)

    return body(xf2d, sec, fst)


def _tc1_body(demb_ref, first_ref, w1t_ref, b1_ref, pool_ref, h_ref, s12_ref):
    demb = demb_ref[...]
    pool = pool_ref[...]
    sum_vec = jnp.dot(demb, pool, preferred_element_type=jnp.float32)
    sumsq = jnp.dot(demb * demb, pool, preferred_element_type=jnp.float32)
    s2 = 0.5 * jnp.sum(sum_vec * sum_vec - sumsq, axis=1)
    s1 = jnp.sum(first_ref[...], axis=1)
    h_ref[...] = (jnp.dot(demb, w1t_ref[...],
                          preferred_element_type=jnp.float32) + b1_ref[...])
    s12_ref[...] = s1 + s2


def _tc2_body(h_ref, s12_ref, g1_ref, be1_ref, w2t_ref, b2_ref, g2_ref,
              be2_ref, bias_ref, out_ref):
    h = h_ref[...]
    m1 = jnp.mean(h, axis=0)
    v1 = jnp.mean((h - m1) ** 2, axis=0)
    h = (h - m1) / jnp.sqrt(v1 + 1e-5) * g1_ref[...] + be1_ref[...]
    h = jnp.maximum(h, 0.0)

    h2 = jnp.dot(h, w2t_ref[...], preferred_element_type=jnp.float32)
    h2 = h2 + b2_ref[...]
    m2 = jnp.mean(h2, axis=0)
    v2 = jnp.mean((h2 - m2) ** 2, axis=0)
    h2 = (h2 - m2) / jnp.sqrt(v2 + 1e-5) * g2_ref[...] + be2_ref[...]
    h2 = jnp.maximum(h2, 0.0)
    sd = jnp.sum(h2, axis=1)

    z = s12_ref[...] + sd + bias_ref[...]
    out_ref[...] = 1.0 / (1.0 + jnp.exp(-z))


def kernel(x, fm_first_w, fm_second_w, w1, b1, g1, be1, w2, b2, g2, be2, bias):
    B, F = x.shape
    total = B * F
    V = fm_second_w.shape[0]
    xf2d = x.reshape(total // 128, 128)
    demb_flat, fvals = _sc_gather(xf2d, fm_second_w, fm_first_w[:, 0], total)
    demb = demb_flat.reshape(B, F * EMB)
    first = fvals.reshape(B, F)

    cb = 2048
    nb = B // cb
    d = F * EMB
    h, s12 = pl.pallas_call(
        _tc1_body,
        grid=(nb,),
        in_specs=[
            pl.BlockSpec((cb, d), lambda i: (i, 0)),
            pl.BlockSpec((cb, F), lambda i: (i, 0)),
            pl.BlockSpec((d, 12), lambda i: (0, 0)),
            pl.BlockSpec((12,), lambda i: (0,)),
            pl.BlockSpec((d, EMB), lambda i: (0, 0)),
        ],
        out_specs=[
            pl.BlockSpec((cb, 12), lambda i: (i, 0)),
            pl.BlockSpec((cb,), lambda i: (i,)),
        ],
        out_shape=[
            jax.ShapeDtypeStruct((B, 12), jnp.float32),
            jax.ShapeDtypeStruct((B,), jnp.float32),
        ],
    )(demb, first, w1.T, b1, jnp.asarray(_POOL))

    out = pl.pallas_call(
        _tc2_body,
        out_shape=jax.ShapeDtypeStruct((B,), jnp.float32),
    )(h, s12, g1, be1, w2.T, b2, g2, be2, bias)
    return out
